# baseline (speedup 1.0000x reference)
import functools

import jax
import jax.numpy as jnp
from jax import lax
from jax.experimental import pallas as pl
from jax.experimental.pallas import tpu as pltpu
from jax.experimental.pallas import tpu_sc as plsc

N = 10000
D = 128
H = 128
E = 320000

NC = 2
NS = 16
L = 16
DA = 144
EBLK = 64
EB = 320
CH = 16
E_PAD = NS * EB * EBLK
NPAD = 10112
RPT = NPAD // NS


def _sc_edge_kernel(xa, tbl, src3, dst3):
  mesh = plsc.VectorSubcoreMesh(core_axis_name="c", subcore_axis_name="s",
                                num_cores=NC, num_subcores=NS)

  @functools.partial(
      pl.kernel,
      out_type=jax.ShapeDtypeStruct((NC, NPAD, DA), jnp.float32),
      mesh=mesh,
      compiler_params=pltpu.CompilerParams(needs_layout_passes=False,
                                           use_tc_tiling_on_sc=False),
      scratch_types=[
          pltpu.VMEM((CH, EBLK), jnp.int32),
          pltpu.VMEM((CH, EBLK), jnp.int32),
          pltpu.VMEM((N,), jnp.int32),
          pltpu.VMEM((EBLK, DA), jnp.float32),
          pltpu.VMEM((EBLK, DA), jnp.float32),
          pltpu.VMEM((EBLK, DA), jnp.float32),
          pltpu.VMEM_SHARED((NPAD, DA), jnp.float32),
          pltpu.SemaphoreType.DMA,
          pltpu.SemaphoreType.DMA,
          pltpu.SemaphoreType.DMA,
      ],
  )
  def k(x_hbm, tbl_hbm, src_hbm, dst_hbm, agg_out,
        sidx, didx, tblv, rows, rows1, rows2, agg_sh, sem, sem1, sem2):
    c = lax.axis_index("c")
    s = lax.axis_index("s")

    pltpu.sync_copy(tbl_hbm.at[c], tblv)

    zv = jnp.zeros((L,), jnp.float32)

    def fill(r, carry):
      for kk in range(DA // L):
        rows[r, pl.ds(kk * L, L)] = zv
        rows1[r, pl.ds(kk * L, L)] = zv
      return carry
    lax.fori_loop(0, EBLK, fill, None)

    base = s * RPT
    for off in range(0, RPT, EBLK):
      nrows = min(EBLK, RPT - off)
      pltpu.sync_copy(rows.at[pl.ds(0, nrows)], agg_sh.at[pl.ds(base + off, nrows)])

    plsc.subcore_barrier()

    bufs = (rows, rows1, rows2)
    sems = (sem, sem1, sem2)

    def _xlate_block(i):
      for kk in range(EBLK // L):
        v = sidx[i, pl.ds(kk * L, L)]
        sidx[i, pl.ds(kk * L, L)] = plsc.load_gather(tblv, [v])

    def chunk(q, carry):
      pltpu.sync_copy(src_hbm.at[c, s, pl.ds(q * CH, CH)], sidx)
      pltpu.sync_copy(dst_hbm.at[c, s, pl.ds(q * CH, CH)], didx)
      _xlate_block(0)
      pend = [None, None, None]
      pend[0] = pltpu.async_copy(x_hbm.at[sidx.at[0]], bufs[0], sems[0])
      _xlate_block(1)
      pend[1] = pltpu.async_copy(x_hbm.at[sidx.at[1]], bufs[1], sems[1])
      for i in range(2, CH):
        _xlate_block(i)
      for i in range(CH):
        pend[i % 3].wait()
        if i + 2 < CH:
          pend[(i + 2) % 3] = pltpu.async_copy(x_hbm.at[sidx.at[i + 2]],
                                               bufs[(i + 2) % 3],
                                               sems[(i + 2) % 3])
        pltpu.sync_copy(bufs[i % 3], agg_sh.at[didx.at[i]], add=True)
      return carry
    lax.fori_loop(0, EB // CH, chunk, None)

    plsc.subcore_barrier()

    pltpu.sync_copy(agg_sh.at[pl.ds(base, RPT)], agg_out.at[c, pl.ds(base, RPT)])

  return k(xa, tbl, src3, dst3)


RB = 1000
NB = N // RB
DEGW = 16


def _tc_loss_body(agg_ref, deg_ref, we_ref, be_ref, wdt_ref, out_ref,
                  sum_acc, ws_ref):
  p = pl.program_id(0)
  j = pl.program_id(1)

  inv = 1.0 / jnp.clip(deg_ref[:, 0:1], 1.0, None)

  @pl.when(p == 0)
  def _():
    @pl.when(j == 0)
    def _():
      sum_acc[...] = jnp.zeros_like(sum_acc)
    pos = jnp.maximum((agg_ref[0] * inv) @ we_ref[...] + be_ref[...], 0.0)
    sum_acc[...] += pos.sum(axis=0, keepdims=True)

  @pl.when(p == 1)
  def _():
    @pl.when(j == 0)
    def _():
      ssum = sum_acc[...] * (1.0 / N)
      ws_ref[...] = jnp.dot(ssum, wdt_ref[...])
      out_ref[0, 0] = 0.0
    ws = ws_ref[...]
    pos = jnp.maximum((agg_ref[0] * inv) @ we_ref[...] + be_ref[...], 0.0)
    neg = jnp.maximum((agg_ref[1] * inv) @ we_ref[...] + be_ref[...], 0.0)
    lp = (pos * ws).sum(axis=1)
    ln = (neg * ws).sum(axis=1)
    c1 = jnp.maximum(lp, 0.0) - lp + jnp.log1p(jnp.exp(-jnp.abs(lp)))
    c2 = jnp.maximum(ln, 0.0) + jnp.log1p(jnp.exp(-jnp.abs(ln)))
    out_ref[0, 0] += (c1.sum() + c2.sum()) * (1.0 / N)


def _tc_loss_kernel(agg, deg, W_enc, b_enc2, W_disc_T):
  return pl.pallas_call(
      _tc_loss_body,
      grid=(2, NB),
      in_specs=[
          pl.BlockSpec((NC, RB, D), lambda p, j: (0, j, 0)),
          pl.BlockSpec((RB, DEGW), lambda p, j: (j, 0)),
          pl.BlockSpec((D, H), lambda p, j: (0, 0)),
          pl.BlockSpec((1, H), lambda p, j: (0, 0)),
          pl.BlockSpec((H, H), lambda p, j: (0, 0)),
      ],
      out_specs=pl.BlockSpec((1, 1), lambda p, j: (0, 0),
                             memory_space=pltpu.SMEM),
      out_shape=jax.ShapeDtypeStruct((1, 1), jnp.float32),
      scratch_shapes=[pltpu.VMEM((1, D), jnp.float32),
                      pltpu.VMEM((1, H), jnp.float32)],
  )(agg, deg, W_enc, b_enc2, W_disc_T)


def kernel(x, edge_index, W_enc, b_enc, W_disc, perm):
  src = edge_index[0].astype(jnp.int32)
  dst = edge_index[1].astype(jnp.int32)
  pad = E_PAD - E
  perm32 = perm.astype(jnp.int32)
  MACRO = 4096
  pad_src = jnp.arange(pad, dtype=jnp.int32) % N
  pad_dst = jnp.arange(pad, dtype=jnp.int32) % (NPAD - N) + N
  sp = jnp.concatenate([src, pad_src])
  dp = jnp.concatenate([dst, pad_dst])

  def macro_sort(key_vals):
    kk = (key_vals >> 8).reshape(-1, MACRO)
    order = jnp.argsort(kk, axis=1, stable=True)
    s2 = jnp.take_along_axis(sp.reshape(-1, MACRO), order, axis=1)
    d2 = jnp.take_along_axis(dp.reshape(-1, MACRO), order, axis=1)
    return s2.reshape(NS, EB, EBLK), d2.reshape(NS, EB, EBLK)

  s0, d0 = macro_sort(sp)
  s1, d1 = macro_sort(perm32[sp])
  src3 = jnp.stack([s0, s1])
  dst3 = jnp.stack([d0, d1])
  tbl = jnp.stack([jnp.arange(N, dtype=jnp.int32), perm32])
  xa = jnp.concatenate(
      [x, jnp.ones((N, 1), jnp.float32), jnp.zeros((N, DA - D - 1), jnp.float32)],
      axis=1)
  agg = _sc_edge_kernel(xa, tbl, src3, dst3)
  feat = agg[:, :, :D]
  deg = agg[0, :, D:D + DEGW]
  out = _tc_loss_kernel(feat, deg, W_enc, b_enc.reshape(1, H), W_disc.T)
  return out[0, 0]


# speedup vs baseline: 7.1972x; 7.1972x over previous
import functools

import jax
import jax.numpy as jnp
from jax import lax
from jax.experimental import pallas as pl
from jax.experimental.pallas import tpu as pltpu
from jax.experimental.pallas import tpu_sc as plsc

N = 10000
D = 128
H = 128
E = 320000

NC = 2
NS = 16
L = 16
DA = 144
EBLK = 32
MACRO = 4096
NBLK = MACRO // EBLK
NMAC = 5
E_PAD = NS * NMAC * MACRO
NPAD = 10112
RPT = NPAD // NS
KB = 40
M14 = (1 << 14) - 1


def _sc_edge_kernel(xa, tblp, pe3):
  mesh = plsc.VectorSubcoreMesh(core_axis_name="c", subcore_axis_name="s",
                                num_cores=NC, num_subcores=NS)

  @functools.partial(
      pl.kernel,
      out_type=jax.ShapeDtypeStruct((NC, NPAD, DA), jnp.float32),
      mesh=mesh,
      compiler_params=pltpu.CompilerParams(needs_layout_passes=False,
                                           use_tc_tiling_on_sc=False),
      scratch_types=[
          pltpu.VMEM((NBLK, EBLK), jnp.int32),
          pltpu.VMEM((NBLK, EBLK), jnp.int32),
          pltpu.VMEM((KB * L,), jnp.int32),
          pltpu.VMEM((N // 2,), jnp.int32),
          pltpu.VMEM((4, EBLK), jnp.int32),
          pltpu.VMEM((4, EBLK), jnp.int32),
          pltpu.VMEM((EBLK, DA), jnp.float32),
          pltpu.VMEM((EBLK, DA), jnp.float32),
          pltpu.VMEM((EBLK, DA), jnp.float32),
          pltpu.VMEM((EBLK, DA), jnp.float32),
          pltpu.VMEM_SHARED((NPAD, DA), jnp.float32),
          pltpu.SemaphoreType.DMA,
          pltpu.SemaphoreType.DMA,
          pltpu.SemaphoreType.DMA,
          pltpu.SemaphoreType.DMA,
      ],
  )
  def k(x_hbm, tbl_hbm, pe_hbm, agg_out,
        pev, es, hist, tblv, srcb, dstb, rows0, rows1, rows2, rows3,
        agg_sh, sem0, sem1, sem2, sem3):
    c = lax.axis_index("c")
    s = lax.axis_index("s")
    bufs = (rows0, rows1, rows2, rows3)
    sems = (sem0, sem1, sem2, sem3)

    pltpu.sync_copy(tbl_hbm.at[c], tblv)

    zv = jnp.zeros((L,), jnp.float32)

    def fill(r, carry):
      for kk in range(DA // L):
        rows0[r, pl.ds(kk * L, L)] = zv
      return carry
    lax.fori_loop(0, EBLK, fill, None)

    base = s * RPT
    for off in range(0, RPT, EBLK):
      nrows = min(EBLK, RPT - off)
      pltpu.sync_copy(rows0.at[pl.ds(0, nrows)],
                      agg_sh.at[pl.ds(base + off, nrows)])

    plsc.subcore_barrier()

    iota16 = lax.iota(jnp.int32, L)
    ones16 = jnp.ones((L,), jnp.int32)

    def macro(m, carry):
      pltpu.sync_copy(pe_hbm.at[s, m], pev)

      for b in range(KB):
        hist[pl.ds(b * L, L)] = jnp.zeros((L,), jnp.int32)

      def thist(i, cr):
        for half in range(EBLK // L):
          sl = pl.ds(half * L, L)
          v = pev[i, sl]
          sv = v & M14
          w = plsc.load_gather(tblv, [sv >> 1])
          t = (w >> ((sv & 1) << 4)) & 0xFFFF
          pev[i, sl] = t | ((v >> 14) << 14)
          key = (t >> 8) * L + iota16
          plsc.addupdate_scatter(hist, [key], ones16)
        return cr
      lax.fori_loop(0, NBLK, thist, None)

      carry_s = jnp.int32(0)
      for b in range(KB):
        vec = hist[pl.ds(b * L, L)]
        incl = plsc.cumsum(vec)
        hist[pl.ds(b * L, L)] = incl - vec + carry_s
        carry_s = carry_s + jnp.sum(vec)

      def place(i, cr):
        for half in range(EBLK // L):
          sl = pl.ds(half * L, L)
          v = pev[i, sl]
          key = ((v & M14) >> 8) * L + iota16
          pos = plsc.load_gather(hist, [key])
          plsc.store_scatter(es, [pos >> 5, pos & (EBLK - 1)], v)
          plsc.store_scatter(hist, [key], pos + 1)
        return cr
      lax.fori_loop(0, NBLK, place, None)

      def unpack_src(j, row):
        for half in range(EBLK // L):
          sl = pl.ds(half * L, L)
          srcb[row, sl] = es[j, sl] & M14

      def unpack_dst(j, row):
        for half in range(EBLK // L):
          sl = pl.ds(half * L, L)
          dstb[row, sl] = es[j, sl] >> 14

      unpack_src(0, 0)
      pltpu.async_copy(x_hbm.at[srcb.at[0]], bufs[0], sems[0])
      unpack_src(1, 1)
      pltpu.async_copy(x_hbm.at[srcb.at[1]], bufs[1], sems[1])

      def ring(i, cr):
        for hh in range(4):
          j = i * 4 + hh
          unpack_dst(j, hh)
          pltpu.make_async_copy(x_hbm.at[srcb.at[hh]], bufs[hh],
                                sems[hh]).wait()

          @pl.when(j + 2 < NBLK)
          def _():
            unpack_src(j + 2, (hh + 2) % 4)
            pltpu.async_copy(x_hbm.at[srcb.at[(hh + 2) % 4]],
                             bufs[(hh + 2) % 4], sems[(hh + 2) % 4])

          pltpu.sync_copy(bufs[hh], agg_sh.at[dstb.at[hh]], add=True)
        return cr
      lax.fori_loop(0, NBLK // 4, ring, None)
      return carry
    lax.fori_loop(0, NMAC, macro, None)

    plsc.subcore_barrier()

    pltpu.sync_copy(agg_sh.at[pl.ds(base, RPT)], agg_out.at[c, pl.ds(base, RPT)])

  return k(xa, tblp, pe3)


RB = 1000
NB = N // RB
DEGW = 16


def _tc_loss_body(agg_ref, deg_ref, we_ref, be_ref, wdt_ref, out_ref,
                  sum_acc, ws_ref):
  p = pl.program_id(0)
  j = pl.program_id(1)

  inv = 1.0 / jnp.clip(deg_ref[:, 0:1], 1.0, None)

  @pl.when(p == 0)
  def _():
    @pl.when(j == 0)
    def _():
      sum_acc[...] = jnp.zeros_like(sum_acc)
    pos = jnp.maximum((agg_ref[0] * inv) @ we_ref[...] + be_ref[...], 0.0)
    sum_acc[...] += pos.sum(axis=0, keepdims=True)

  @pl.when(p == 1)
  def _():
    @pl.when(j == 0)
    def _():
      ssum = sum_acc[...] * (1.0 / N)
      ws_ref[...] = jnp.dot(ssum, wdt_ref[...])
      out_ref[0, 0] = 0.0
    ws = ws_ref[...]
    pos = jnp.maximum((agg_ref[0] * inv) @ we_ref[...] + be_ref[...], 0.0)
    neg = jnp.maximum((agg_ref[1] * inv) @ we_ref[...] + be_ref[...], 0.0)
    lp = (pos * ws).sum(axis=1)
    ln = (neg * ws).sum(axis=1)
    c1 = jnp.maximum(lp, 0.0) - lp + jnp.log1p(jnp.exp(-jnp.abs(lp)))
    c2 = jnp.maximum(ln, 0.0) + jnp.log1p(jnp.exp(-jnp.abs(ln)))
    out_ref[0, 0] += (c1.sum() + c2.sum()) * (1.0 / N)


def _tc_loss_kernel(agg, deg, W_enc, b_enc2, W_disc_T):
  return pl.pallas_call(
      _tc_loss_body,
      grid=(2, NB),
      in_specs=[
          pl.BlockSpec((NC, RB, D), lambda p, j: (0, j, 0)),
          pl.BlockSpec((RB, DEGW), lambda p, j: (j, 0)),
          pl.BlockSpec((D, H), lambda p, j: (0, 0)),
          pl.BlockSpec((1, H), lambda p, j: (0, 0)),
          pl.BlockSpec((H, H), lambda p, j: (0, 0)),
      ],
      out_specs=pl.BlockSpec((1, 1), lambda p, j: (0, 0),
                             memory_space=pltpu.SMEM),
      out_shape=jax.ShapeDtypeStruct((1, 1), jnp.float32),
      scratch_shapes=[pltpu.VMEM((1, D), jnp.float32),
                      pltpu.VMEM((1, H), jnp.float32)],
  )(agg, deg, W_enc, b_enc2, W_disc_T)


def kernel(x, edge_index, W_enc, b_enc, W_disc, perm):
  src = edge_index[0].astype(jnp.int32)
  dst = edge_index[1].astype(jnp.int32)
  pad = E_PAD - E
  perm32 = perm.astype(jnp.int32)
  pad_src = jnp.arange(pad, dtype=jnp.int32) % N
  pad_dst = jnp.arange(pad, dtype=jnp.int32) % (NPAD - N) + N
  sp = jnp.concatenate([src, pad_src])
  dp = jnp.concatenate([dst, pad_dst])
  pe3 = (sp | (dp << 14)).reshape(NS, NMAC, NBLK, EBLK)
  tbl = jnp.stack([jnp.arange(N, dtype=jnp.int32), perm32])
  tblp = tbl[:, 0::2] | (tbl[:, 1::2] << 16)
  xa = jnp.concatenate(
      [x, jnp.ones((N, 1), jnp.float32), jnp.zeros((N, DA - D - 1), jnp.float32)],
      axis=1)
  agg = _sc_edge_kernel(xa, tblp, pe3)
  feat = agg[:, :, :D]
  deg = agg[0, :, D:D + DEGW]
  out = _tc_loss_kernel(feat, deg, W_enc, b_enc.reshape(1, H), W_disc.T)
  return out[0, 0]


# speedup vs baseline: 7.3513x; 1.0214x over previous
import functools

import jax
import jax.numpy as jnp
from jax import lax
from jax.experimental import pallas as pl
from jax.experimental.pallas import tpu as pltpu
from jax.experimental.pallas import tpu_sc as plsc

N = 10000
D = 128
H = 128
E = 320000

NC = 2
NS = 16
L = 16
DA = 144
EBLK = 32
MACRO = 4096
NBLK = MACRO // EBLK
NMAC = 5
E_PAD = NS * NMAC * MACRO
NPAD = 10112
RPT = NPAD // NS
KB = 40
M14 = (1 << 14) - 1


def _sc_edge_kernel(xa, tblp, pe3):
  mesh = plsc.VectorSubcoreMesh(core_axis_name="c", subcore_axis_name="s",
                                num_cores=NC, num_subcores=NS)

  @functools.partial(
      pl.kernel,
      out_type=jax.ShapeDtypeStruct((NC, NPAD, DA), jnp.float32),
      mesh=mesh,
      compiler_params=pltpu.CompilerParams(needs_layout_passes=False,
                                           use_tc_tiling_on_sc=False),
      scratch_types=[
          pltpu.VMEM((NBLK, EBLK), jnp.int32),
          pltpu.VMEM((NBLK, EBLK), jnp.int32),
          pltpu.VMEM((KB * L,), jnp.int32),
          pltpu.VMEM((N // 2,), jnp.int32),
          pltpu.VMEM((4, EBLK), jnp.int32),
          pltpu.VMEM((4, EBLK), jnp.int32),
          pltpu.VMEM((EBLK, DA), jnp.float32),
          pltpu.VMEM((EBLK, DA), jnp.float32),
          pltpu.VMEM((EBLK, DA), jnp.float32),
          pltpu.VMEM((EBLK, DA), jnp.float32),
          pltpu.VMEM_SHARED((NPAD, DA), jnp.float32),
          pltpu.SemaphoreType.DMA,
          pltpu.SemaphoreType.DMA,
          pltpu.SemaphoreType.DMA,
          pltpu.SemaphoreType.DMA,
      ],
  )
  def k(x_hbm, tbl_hbm, pe_hbm, agg_out,
        pev, es, hist, tblv, srcb, dstb, rows0, rows1, rows2, rows3,
        agg_sh, sem0, sem1, sem2, sem3):
    c = lax.axis_index("c")
    s = lax.axis_index("s")
    bufs = (rows0, rows1, rows2, rows3)
    sems = (sem0, sem1, sem2, sem3)

    pltpu.sync_copy(tbl_hbm.at[c], tblv)

    zv = jnp.zeros((L,), jnp.float32)

    def fill(r, carry):
      for kk in range(DA // L):
        rows0[r, pl.ds(kk * L, L)] = zv
      return carry
    lax.fori_loop(0, EBLK, fill, None)

    base = s * RPT
    for off in range(0, RPT, EBLK):
      nrows = min(EBLK, RPT - off)
      pltpu.sync_copy(rows0.at[pl.ds(0, nrows)],
                      agg_sh.at[pl.ds(base + off, nrows)])

    plsc.subcore_barrier()

    iota16 = lax.iota(jnp.int32, L)
    ones16 = jnp.ones((L,), jnp.int32)

    def macro(m, carry):
      pltpu.sync_copy(pe_hbm.at[s, m], pev)

      for b in range(KB):
        hist[pl.ds(b * L, L)] = jnp.zeros((L,), jnp.int32)

      def thist(i, cr):
        for half in range(EBLK // L):
          sl = pl.ds(half * L, L)
          v = pev[i, sl]
          sv = v & M14
          w = plsc.load_gather(tblv, [sv >> 1])
          t = (w >> ((sv & 1) << 4)) & 0xFFFF
          pev[i, sl] = t | ((v >> 14) << 14)
          key = (t >> 8) * L + iota16
          plsc.addupdate_scatter(hist, [key], ones16)
        return cr
      lax.fori_loop(0, NBLK, thist, None)

      carry_s = jnp.int32(0)
      for b in range(KB):
        vec = hist[pl.ds(b * L, L)]
        incl = plsc.cumsum(vec)
        hist[pl.ds(b * L, L)] = incl - vec + carry_s
        carry_s = carry_s + jnp.sum(vec)

      def place(i, cr):
        for half in range(EBLK // L):
          sl = pl.ds(half * L, L)
          v = pev[i, sl]
          key = ((v & M14) >> 8) * L + iota16
          pos = plsc.load_gather(hist, [key])
          plsc.store_scatter(es, [pos >> 5, pos & (EBLK - 1)], v)
          plsc.store_scatter(hist, [key], pos + 1)
        return cr
      lax.fori_loop(0, NBLK, place, None)

      def unpack_src(j, row):
        for half in range(EBLK // L):
          sl = pl.ds(half * L, L)
          srcb[row, sl] = es[j, sl] & M14

      def unpack_dst(j, row):
        for half in range(EBLK // L):
          sl = pl.ds(half * L, L)
          dstb[row, sl] = es[j, sl] >> 14

      unpack_src(0, 0)
      pltpu.async_copy(x_hbm.at[srcb.at[0]], bufs[0], sems[0])
      unpack_src(1, 1)
      pltpu.async_copy(x_hbm.at[srcb.at[1]], bufs[1], sems[1])

      def ring(i, cr):
        for hh in range(4):
          j = i * 4 + hh
          unpack_dst(j, hh)
          pltpu.make_async_copy(x_hbm.at[srcb.at[hh]], bufs[hh],
                                sems[hh]).wait()

          @pl.when(j + 2 < NBLK)
          def _():
            unpack_src(j + 2, (hh + 2) % 4)
            pltpu.async_copy(x_hbm.at[srcb.at[(hh + 2) % 4]],
                             bufs[(hh + 2) % 4], sems[(hh + 2) % 4])

          pltpu.sync_copy(bufs[hh], agg_sh.at[dstb.at[hh]], add=True)
        return cr
      lax.fori_loop(0, NBLK // 4, ring, None)
      return carry
    lax.fori_loop(0, NMAC, macro, None)

    plsc.subcore_barrier()

    pltpu.sync_copy(agg_sh.at[pl.ds(base, RPT)], agg_out.at[c, pl.ds(base, RPT)])

  return k(xa, tblp, pe3)


RB = 1000
NB = N // RB
DEGW = 16


def _tc_loss_body(agg_ref, we_ref, be_ref, wdt_ref, out_ref,
                  sum_acc, ws_ref):
  p = pl.program_id(0)
  j = pl.program_id(1)

  inv = 1.0 / jnp.clip(agg_ref[0, :, D:D + 1], 1.0, None)

  @pl.when(p == 0)
  def _():
    @pl.when(j == 0)
    def _():
      sum_acc[...] = jnp.zeros_like(sum_acc)
    pos = jnp.maximum((agg_ref[0, :, :D] * inv) @ we_ref[...] + be_ref[...], 0.0)
    sum_acc[...] += pos.sum(axis=0, keepdims=True)

  @pl.when(p == 1)
  def _():
    @pl.when(j == 0)
    def _():
      ssum = sum_acc[...] * (1.0 / N)
      ws_ref[...] = jnp.dot(ssum, wdt_ref[...])
      out_ref[0, 0] = 0.0
    ws = ws_ref[...]
    pos = jnp.maximum((agg_ref[0, :, :D] * inv) @ we_ref[...] + be_ref[...], 0.0)
    neg = jnp.maximum((agg_ref[1, :, :D] * inv) @ we_ref[...] + be_ref[...], 0.0)
    lp = (pos * ws).sum(axis=1)
    ln = (neg * ws).sum(axis=1)
    c1 = jnp.maximum(lp, 0.0) - lp + jnp.log1p(jnp.exp(-jnp.abs(lp)))
    c2 = jnp.maximum(ln, 0.0) + jnp.log1p(jnp.exp(-jnp.abs(ln)))
    out_ref[0, 0] += (c1.sum() + c2.sum()) * (1.0 / N)


def _tc_loss_kernel(agg, W_enc, b_enc2, W_disc_T):
  return pl.pallas_call(
      _tc_loss_body,
      grid=(2, NB),
      in_specs=[
          pl.BlockSpec((NC, RB, DA), lambda p, j: (0, j, 0)),
          pl.BlockSpec((D, H), lambda p, j: (0, 0)),
          pl.BlockSpec((1, H), lambda p, j: (0, 0)),
          pl.BlockSpec((H, H), lambda p, j: (0, 0)),
      ],
      out_specs=pl.BlockSpec((1, 1), lambda p, j: (0, 0),
                             memory_space=pltpu.SMEM),
      out_shape=jax.ShapeDtypeStruct((1, 1), jnp.float32),
      scratch_shapes=[pltpu.VMEM((1, D), jnp.float32),
                      pltpu.VMEM((1, H), jnp.float32)],
  )(agg, W_enc, b_enc2, W_disc_T)


def kernel(x, edge_index, W_enc, b_enc, W_disc, perm):
  src = edge_index[0].astype(jnp.int32)
  dst = edge_index[1].astype(jnp.int32)
  pad = E_PAD - E
  perm32 = perm.astype(jnp.int32)
  pad_src = jnp.arange(pad, dtype=jnp.int32) % N
  pad_dst = jnp.arange(pad, dtype=jnp.int32) % (NPAD - N) + N
  sp = jnp.concatenate([src, pad_src])
  dp = jnp.concatenate([dst, pad_dst])
  pe3 = (sp | (dp << 14)).reshape(NS, NMAC, NBLK, EBLK)
  tbl = jnp.stack([jnp.arange(N, dtype=jnp.int32), perm32])
  tblp = tbl[:, 0::2] | (tbl[:, 1::2] << 16)
  xa = jnp.concatenate(
      [x, jnp.ones((N, 1), jnp.float32), jnp.zeros((N, DA - D - 1), jnp.float32)],
      axis=1)
  agg = _sc_edge_kernel(xa, tblp, pe3)
  out = _tc_loss_kernel(agg, W_enc, b_enc.reshape(1, H), W_disc.T)
  return out[0, 0]
